# baseline (speedup 1.0000x reference)
import jax
import jax.numpy as jnp
from jax import lax
from jax.experimental import pallas as pl
from jax.experimental.pallas import tpu as pltpu

_RATIO = 0.7
_H = 512
_W = 512
_BLK = 6
_N = 96 // _BLK
_P = 8
_RS = 504
_CS = 384


def _in_copy(img_hbm, buf, sem_in, k, base):
    return pltpu.make_async_copy(
        img_hbm.at[pl.ds(base, _BLK)], buf.at[k], sem_in.at[k]
    )


def _out_copy(out_hbm, buf, sem_out, k, base):
    return pltpu.make_async_copy(
        buf.at[k], out_hbm.at[pl.ds(base, _BLK)], sem_out.at[k]
    )


def _region(t, l, b, r, shape, roff, coff):
    row_id = lax.broadcasted_iota(jnp.int32, shape, 0) + roff
    col_id = lax.broadcasted_iota(jnp.int32, shape, 1) + coff
    return (row_id >= t) & (row_id < b) & (col_id >= l) & (col_id < r)


def _body(mask_hbm, img_hbm, out_hbm, bbox, mvm, buf, sem_in, sem_out, msem):
    i = pl.program_id(0)

    @pl.when(i == 0)
    def _():
        pltpu.make_async_copy(mask_hbm, mvm, msem).start()

    for k in range(_P):
        @pl.when((i >= _P) & (i < _N) & (lax.rem(i, _P) == k))
        def _(k=k):
            _out_copy(out_hbm, buf, sem_out, k, (i - _P) * _BLK).wait()

        @pl.when((i < _N) & (lax.rem(i, _P) == k))
        def _(k=k):
            _in_copy(img_hbm, buf, sem_in, k, i * _BLK).start()

    @pl.when(i == 0)
    def _():
        pltpu.make_async_copy(mask_hbm, mvm, msem).wait()
        m = mvm[...]
        nz = m != 0.0
        row_id = lax.broadcasted_iota(jnp.int32, (_H, _W), 0)
        col_id = lax.broadcasted_iota(jnp.int32, (_H, _W), 1)
        top = jnp.min(jnp.where(nz, row_id, _H))
        bottom = jnp.max(jnp.where(nz, row_id, -1))
        left = jnp.min(jnp.where(nz, col_id, _W))
        right = jnp.max(jnp.where(nz, col_id, -1))
        bbox[0] = jnp.floor(top * _RATIO).astype(jnp.int32)
        bbox[1] = jnp.floor(left * _RATIO).astype(jnp.int32)
        bbox[2] = jnp.floor(bottom + (_H - bottom) * (1.0 - _RATIO)).astype(jnp.int32)
        bbox[3] = jnp.floor(right + (_W - right) * (1.0 - _RATIO)).astype(jnp.int32)

    t = bbox[0]
    l = bbox[1]
    b = bbox[2]
    r = bbox[3]
    for k in range(_P):
        @pl.when((i >= 1) & (lax.rem(i - 1, _P) == k))
        def _(k=k):
            base = (i - 1) * _BLK
            _in_copy(img_hbm, buf, sem_in, k, base).wait()

            strips_ok = (t <= 0) & (l <= 0) & (b >= _RS) & (r >= _CS)

            @pl.when(strips_ok)
            def _():
                regc = _region(t, l, b, r, (_H, _W - _CS), 0, _CS)
                buf[k, :, :, _CS:] = jnp.where(
                    regc[None, :, :], buf[k, :, :, _CS:], -1.0
                )
                regr = _region(t, l, b, r, (_H - _RS, _CS), _RS, 0)
                buf[k, :, _RS:, :_CS] = jnp.where(
                    regr[None, :, :], buf[k, :, _RS:, :_CS], -1.0
                )

            @pl.when(jnp.logical_not(strips_ok))
            def _():
                regf = _region(t, l, b, r, (_H, _W), 0, 0)
                buf[k] = jnp.where(regf[None, :, :], buf[k], -1.0)

            _out_copy(out_hbm, buf, sem_out, k, base).start()

        jlast = max(j for j in range(_N) if j % _P == k and j + _P >= _N)

        @pl.when(i == _N)
        def _(k=k, jlast=jlast):
            _out_copy(out_hbm, buf, sem_out, k, jlast * _BLK).wait()


@jax.jit
def kernel(image, cover, mask):
    del cover
    m = mask[0, 0]
    n = image.shape[0] * image.shape[1]
    x = image.reshape(n, _H, _W)
    out = pl.pallas_call(
        _body,
        grid=(_N + 1,),
        in_specs=[
            pl.BlockSpec(memory_space=pl.ANY),
            pl.BlockSpec(memory_space=pl.ANY),
        ],
        out_specs=pl.BlockSpec(memory_space=pl.ANY),
        out_shape=jax.ShapeDtypeStruct((n, _H, _W), jnp.float32),
        scratch_shapes=[
            pltpu.SMEM((4,), jnp.int32),
            pltpu.VMEM((_H, _W), jnp.float32),
            pltpu.VMEM((_P, _BLK, _H, _W), jnp.float32),
            pltpu.SemaphoreType.DMA((_P,)),
            pltpu.SemaphoreType.DMA((_P,)),
            pltpu.SemaphoreType.DMA,
        ],
        compiler_params=pltpu.CompilerParams(
            dimension_semantics=("arbitrary",),
        ),
    )(m, x)
    return out.reshape(image.shape)


# speedup vs baseline: 1.0048x; 1.0048x over previous
import jax
import jax.numpy as jnp
from jax import lax
from jax.experimental import pallas as pl
from jax.experimental.pallas import tpu as pltpu

_RATIO = 0.7
_H = 512
_W = 512
_BLK = 8
_N = 96 // _BLK
_P = 6
_RS = 504
_CS = 384


def _in_copy(img_hbm, buf, sem_in, k, base):
    return pltpu.make_async_copy(
        img_hbm.at[pl.ds(base, _BLK)], buf.at[k], sem_in.at[k]
    )


def _out_copy(out_hbm, buf, sem_out, k, base):
    return pltpu.make_async_copy(
        buf.at[k], out_hbm.at[pl.ds(base, _BLK)], sem_out.at[k]
    )


def _region(t, l, b, r, shape, roff, coff):
    row_id = lax.broadcasted_iota(jnp.int32, shape, 0) + roff
    col_id = lax.broadcasted_iota(jnp.int32, shape, 1) + coff
    return (row_id >= t) & (row_id < b) & (col_id >= l) & (col_id < r)


def _body(mask_hbm, img_hbm, out_hbm, bbox, mvm, buf, sem_in, sem_out, msem):
    i = pl.program_id(0)

    @pl.when(i == 0)
    def _():
        pltpu.make_async_copy(mask_hbm, mvm, msem).start()

    for k in range(_P):
        @pl.when((i >= _P) & (i < _N) & (lax.rem(i, _P) == k))
        def _(k=k):
            _out_copy(out_hbm, buf, sem_out, k, (i - _P) * _BLK).wait()

        @pl.when((i < _N) & (lax.rem(i, _P) == k))
        def _(k=k):
            _in_copy(img_hbm, buf, sem_in, k, i * _BLK).start()

    @pl.when(i == 0)
    def _():
        pltpu.make_async_copy(mask_hbm, mvm, msem).wait()
        m = mvm[...]
        nz = m != 0.0
        row_id = lax.broadcasted_iota(jnp.int32, (_H, _W), 0)
        col_id = lax.broadcasted_iota(jnp.int32, (_H, _W), 1)
        top = jnp.min(jnp.where(nz, row_id, _H))
        bottom = jnp.max(jnp.where(nz, row_id, -1))
        left = jnp.min(jnp.where(nz, col_id, _W))
        right = jnp.max(jnp.where(nz, col_id, -1))
        bbox[0] = jnp.floor(top * _RATIO).astype(jnp.int32)
        bbox[1] = jnp.floor(left * _RATIO).astype(jnp.int32)
        bbox[2] = jnp.floor(bottom + (_H - bottom) * (1.0 - _RATIO)).astype(jnp.int32)
        bbox[3] = jnp.floor(right + (_W - right) * (1.0 - _RATIO)).astype(jnp.int32)

    t = bbox[0]
    l = bbox[1]
    b = bbox[2]
    r = bbox[3]
    for k in range(_P):
        @pl.when((i >= 1) & (lax.rem(i - 1, _P) == k))
        def _(k=k):
            base = (i - 1) * _BLK
            _in_copy(img_hbm, buf, sem_in, k, base).wait()

            strips_ok = (t <= 0) & (l <= 0) & (b >= _RS) & (r >= _CS)

            @pl.when(strips_ok)
            def _():
                regc = _region(t, l, b, r, (_H, _W - _CS), 0, _CS)
                buf[k, :, :, _CS:] = jnp.where(
                    regc[None, :, :], buf[k, :, :, _CS:], -1.0
                )
                regr = _region(t, l, b, r, (_H - _RS, _CS), _RS, 0)
                buf[k, :, _RS:, :_CS] = jnp.where(
                    regr[None, :, :], buf[k, :, _RS:, :_CS], -1.0
                )

            @pl.when(jnp.logical_not(strips_ok))
            def _():
                regf = _region(t, l, b, r, (_H, _W), 0, 0)
                buf[k] = jnp.where(regf[None, :, :], buf[k], -1.0)

            _out_copy(out_hbm, buf, sem_out, k, base).start()

        jlast = max(j for j in range(_N) if j % _P == k and j + _P >= _N)

        @pl.when(i == _N)
        def _(k=k, jlast=jlast):
            _out_copy(out_hbm, buf, sem_out, k, jlast * _BLK).wait()


@jax.jit
def kernel(image, cover, mask):
    del cover
    m = mask[0, 0]
    n = image.shape[0] * image.shape[1]
    x = image.reshape(n, _H, _W)
    out = pl.pallas_call(
        _body,
        grid=(_N + 1,),
        in_specs=[
            pl.BlockSpec(memory_space=pl.ANY),
            pl.BlockSpec(memory_space=pl.ANY),
        ],
        out_specs=pl.BlockSpec(memory_space=pl.ANY),
        out_shape=jax.ShapeDtypeStruct((n, _H, _W), jnp.float32),
        scratch_shapes=[
            pltpu.SMEM((4,), jnp.int32),
            pltpu.VMEM((_H, _W), jnp.float32),
            pltpu.VMEM((_P, _BLK, _H, _W), jnp.float32),
            pltpu.SemaphoreType.DMA((_P,)),
            pltpu.SemaphoreType.DMA((_P,)),
            pltpu.SemaphoreType.DMA,
        ],
        compiler_params=pltpu.CompilerParams(
            dimension_semantics=("arbitrary",),
        ),
    )(m, x)
    return out.reshape(image.shape)
